# baseline (speedup 1.0000x reference)
import functools

import jax
import jax.numpy as jnp
from jax import lax
from jax.experimental import pallas as pl
from jax.experimental.pallas import tpu as pltpu
from jax.experimental.pallas import tpu_sc as plsc

_INFO = plsc.get_sparse_core_info()
_NC = _INFO.num_cores
_NS = _INFO.num_subcores
_LANES = _INFO.num_lanes
_NW = _NC * _NS


@functools.lru_cache(maxsize=None)
def _build(n_rows: int, d: int, chunk: int):
    rows_per_worker = n_rows // _NW
    n_chunks = rows_per_worker // chunk
    mesh = plsc.VectorSubcoreMesh(core_axis_name="c", subcore_axis_name="s")
    scale = float(d) ** 0.5

    @functools.partial(
        pl.kernel,
        mesh=mesh,
        out_type=jax.ShapeDtypeStruct((n_rows, d), jnp.float32),
        scratch_types=[
            pltpu.VMEM((n_chunks, chunk), jnp.int32),
            pltpu.VMEM((chunk, d), jnp.float32),
            pltpu.VMEM((chunk, d), jnp.float32),
            pltpu.SemaphoreType.DMA,
        ],
    )
    def sc_kernel(word_hbm, idx_hbm, table_hbm, out_hbm, idx_v, word_v, rows_v, sem):
        wid = lax.axis_index("s") * _NC + lax.axis_index("c")
        pltpu.sync_copy(idx_hbm.at[wid], idx_v)

        def chunk_body(j, carry):
            base = wid * rows_per_worker + j * chunk
            gather = pltpu.async_copy(table_hbm.at[idx_v.at[j]], rows_v, sem)
            pltpu.sync_copy(word_hbm.at[pl.ds(base, chunk)], word_v)
            gather.wait()

            def row_body(r, c):
                for k in range(d // _LANES):
                    sl = pl.ds(k * _LANES, _LANES)
                    word_v[r, sl] = word_v[r, sl] * scale + rows_v[r, sl]
                return c

            lax.fori_loop(0, chunk, row_body, 0)
            pltpu.sync_copy(word_v, out_hbm.at[pl.ds(base, chunk)])
            return carry

        lax.fori_loop(0, n_chunks, chunk_body, 0)

    return sc_kernel


def kernel(src_word, src_pos, emb_table):
    b, s, d = src_word.shape
    n_rows = b * s
    chunk = 32
    rows_per_worker = n_rows // _NW
    n_chunks = rows_per_worker // chunk
    word = src_word.reshape(n_rows, d)
    idx = src_pos.astype(jnp.int32).reshape(_NW, n_chunks, chunk)
    out = _build(n_rows, d, chunk)(word, idx, emb_table)
    return out.reshape(b, s, d)


# speedup vs baseline: 1.5237x; 1.5237x over previous
import functools

import jax
import jax.numpy as jnp
from jax import lax
from jax.experimental import pallas as pl
from jax.experimental.pallas import tpu as pltpu
from jax.experimental.pallas import tpu_sc as plsc

_INFO = plsc.get_sparse_core_info()
_NC = _INFO.num_cores
_NS = _INFO.num_subcores
_LANES = _INFO.num_lanes
_NW = _NC * _NS


@functools.lru_cache(maxsize=None)
def _build(n_rows: int, d: int, chunk: int):
    rows_per_worker = n_rows // _NW
    n_chunks = rows_per_worker // chunk
    assert n_chunks % 2 == 0
    spr = d // _LANES
    shift = spr.bit_length() - 1
    assert 1 << shift == spr
    mesh = plsc.VectorSubcoreMesh(core_axis_name="c", subcore_axis_name="s")
    scale = float(d) ** 0.5

    @functools.partial(
        pl.kernel,
        mesh=mesh,
        out_type=jax.ShapeDtypeStruct((n_rows, d), jnp.float32),
        scratch_types=[
            pltpu.VMEM((n_chunks, chunk), jnp.int32),
            pltpu.VMEM((2, chunk, d), jnp.float32),
            pltpu.VMEM((2, chunk, d), jnp.float32),
            pltpu.VMEM((2, chunk, d), jnp.float32),
            pltpu.SemaphoreType.DMA,
            pltpu.SemaphoreType.DMA,
            pltpu.SemaphoreType.DMA,
            pltpu.SemaphoreType.DMA,
            pltpu.SemaphoreType.DMA,
            pltpu.SemaphoreType.DMA,
        ],
    )
    def sc_kernel(word_hbm, idx_hbm, table_hbm, out_hbm,
                  idx_v, word_v, rows_v, outb_v,
                  gs0, gs1, ws0, ws1, os0, os1):
        gs = (gs0, gs1)
        ws = (ws0, ws1)
        osm = (os0, os1)
        wid = lax.axis_index("s") * _NC + lax.axis_index("c")
        row0 = wid * rows_per_worker
        pltpu.sync_copy(idx_hbm.at[wid], idx_v)

        def issue_inputs(j, p):
            pltpu.async_copy(table_hbm.at[idx_v.at[j]], rows_v.at[p], gs[p])
            base = row0 + j * chunk
            pltpu.async_copy(word_hbm.at[pl.ds(base, chunk)], word_v.at[p], ws[p])

        issue_inputs(0, 0)

        def pair_body(i, carry):
            for b in (0, 1):
                j = 2 * i + b
                jn = lax.rem(j + 1, n_chunks)
                issue_inputs(jn, 1 - b)
                pltpu.make_async_copy(
                    table_hbm.at[idx_v.at[j]], rows_v.at[b], gs[b]).wait()
                base = row0 + j * chunk
                pltpu.make_async_copy(
                    word_hbm.at[pl.ds(base, chunk)], word_v.at[b], ws[b]).wait()

                @pl.when(i >= 1)
                def _():
                    pltpu.make_async_copy(
                        outb_v.at[b], out_hbm.at[pl.ds(base, chunk)],
                        osm[b]).wait()

                @plsc.parallel_loop(0, chunk * spr, unroll=4)
                def _(ii):
                    r = ii >> shift
                    sl = pl.ds((ii - (r << shift)) * _LANES, _LANES)
                    outb_v[b, r, sl] = (word_v[b, r, sl] * scale
                                        + rows_v[b, r, sl])

                pltpu.async_copy(
                    outb_v.at[b], out_hbm.at[pl.ds(base, chunk)], osm[b])
            return carry

        lax.fori_loop(0, n_chunks // 2, pair_body, 0)

        for b, jlast in ((0, n_chunks - 2), (1, n_chunks - 1)):
            last_base = row0 + jlast * chunk
            pltpu.make_async_copy(
                outb_v.at[b], out_hbm.at[pl.ds(last_base, chunk)],
                osm[b]).wait()
        pltpu.make_async_copy(
            table_hbm.at[idx_v.at[0]], rows_v.at[0], gs[0]).wait()
        pltpu.make_async_copy(
            word_hbm.at[pl.ds(row0, chunk)], word_v.at[0], ws[0]).wait()

    return sc_kernel


def kernel(src_word, src_pos, emb_table):
    b, s, d = src_word.shape
    n_rows = b * s
    chunk = 16
    rows_per_worker = n_rows // _NW
    n_chunks = rows_per_worker // chunk
    word = src_word.reshape(n_rows, d)
    idx = src_pos.astype(jnp.int32).reshape(_NW, n_chunks, chunk)
    out = _build(n_rows, d, chunk)(word, idx, emb_table)
    return out.reshape(b, s, d)


# speedup vs baseline: 1.5288x; 1.0034x over previous
import functools

import jax
import jax.numpy as jnp
from jax import lax
from jax.experimental import pallas as pl
from jax.experimental.pallas import tpu as pltpu
from jax.experimental.pallas import tpu_sc as plsc

_INFO = plsc.get_sparse_core_info()
_NC = _INFO.num_cores
_NS = _INFO.num_subcores
_LANES = _INFO.num_lanes
_NW = _NC * _NS


@functools.lru_cache(maxsize=None)
def _build(n_rows: int, d: int, chunk: int):
    rows_per_worker = n_rows // _NW
    n_chunks = rows_per_worker // chunk
    assert n_chunks % 2 == 0
    spr = d // _LANES
    shift = spr.bit_length() - 1
    assert 1 << shift == spr
    mesh = plsc.VectorSubcoreMesh(core_axis_name="c", subcore_axis_name="s")
    scale = float(d) ** 0.5

    @functools.partial(
        pl.kernel,
        mesh=mesh,
        out_type=jax.ShapeDtypeStruct((n_rows, d), jnp.float32),
        scratch_types=[
            pltpu.VMEM((n_chunks, chunk), jnp.int32),
            pltpu.VMEM((2, chunk, d), jnp.float32),
            pltpu.VMEM((2, chunk, d), jnp.float32),
            pltpu.VMEM((2, chunk, d), jnp.float32),
            pltpu.SemaphoreType.DMA,
            pltpu.SemaphoreType.DMA,
            pltpu.SemaphoreType.DMA,
            pltpu.SemaphoreType.DMA,
            pltpu.SemaphoreType.DMA,
            pltpu.SemaphoreType.DMA,
        ],
    )
    def sc_kernel(word_hbm, idx_hbm, table_hbm, out_hbm,
                  idx_v, word_v, rows_v, outb_v,
                  gs0, gs1, ws0, ws1, os0, os1):
        gs = (gs0, gs1)
        ws = (ws0, ws1)
        osm = (os0, os1)
        wid = lax.axis_index("s") * _NC + lax.axis_index("c")
        row0 = wid * rows_per_worker
        pltpu.sync_copy(idx_hbm.at[wid], idx_v)

        def issue_inputs(j, p):
            pltpu.async_copy(table_hbm.at[idx_v.at[j]], rows_v.at[p], gs[p])
            base = row0 + j * chunk
            pltpu.async_copy(word_hbm.at[pl.ds(base, chunk)], word_v.at[p], ws[p])

        issue_inputs(0, 0)

        def pair_body(i, carry):
            for b in (0, 1):
                j = 2 * i + b

                @pl.when(j + 1 < n_chunks)
                def _():
                    issue_inputs(j + 1, 1 - b)
                pltpu.make_async_copy(
                    table_hbm.at[idx_v.at[j]], rows_v.at[b], gs[b]).wait()
                base = row0 + j * chunk
                pltpu.make_async_copy(
                    word_hbm.at[pl.ds(base, chunk)], word_v.at[b], ws[b]).wait()

                @pl.when(i >= 1)
                def _():
                    pltpu.make_async_copy(
                        outb_v.at[b], out_hbm.at[pl.ds(base, chunk)],
                        osm[b]).wait()

                @plsc.parallel_loop(0, chunk * spr, unroll=4)
                def _(ii):
                    r = ii >> shift
                    sl = pl.ds((ii - (r << shift)) * _LANES, _LANES)
                    outb_v[b, r, sl] = (word_v[b, r, sl] * scale
                                        + rows_v[b, r, sl])

                pltpu.async_copy(
                    outb_v.at[b], out_hbm.at[pl.ds(base, chunk)], osm[b])
            return carry

        lax.fori_loop(0, n_chunks // 2, pair_body, 0)

        for b, jlast in ((0, n_chunks - 2), (1, n_chunks - 1)):
            last_base = row0 + jlast * chunk
            pltpu.make_async_copy(
                outb_v.at[b], out_hbm.at[pl.ds(last_base, chunk)],
                osm[b]).wait()

    return sc_kernel


def kernel(src_word, src_pos, emb_table):
    b, s, d = src_word.shape
    n_rows = b * s
    chunk = 16
    rows_per_worker = n_rows // _NW
    n_chunks = rows_per_worker // chunk
    word = src_word.reshape(n_rows, d)
    idx = src_pos.astype(jnp.int32).reshape(_NW, n_chunks, chunk)
    out = _build(n_rows, d, chunk)(word, idx, emb_table)
    return out.reshape(b, s, d)
